# baseline (speedup 1.0000x reference)
import functools

import jax
import jax.numpy as jnp
from jax.experimental import pallas as pl
from jax.experimental.pallas import tpu as pltpu

AUG_T = 1024
B, L, D = 16, 4096, 128
BL = 2048


def _body(lens_ref, x_ref, o_ref):
    l = pl.program_id(1)
    slen = lens_ref[pl.program_id(0)]
    pos = jax.lax.broadcasted_iota(jnp.int32, (1, BL, D), 1) + l * BL
    mask = (pos % 10 == 0) & (pos < slen) & (slen > AUG_T)
    o_ref[...] = jnp.where(mask, 0.0, x_ref[...])


def kernel(sequences, seq_lens):
    out = pl.pallas_call(
        _body,
        grid=(B, L // BL),
        in_specs=[
            pl.BlockSpec(memory_space=pltpu.SMEM),
            pl.BlockSpec((1, BL, D), lambda b, l: (b, l, 0)),
        ],
        out_specs=pl.BlockSpec((1, BL, D), lambda b, l: (b, l, 0)),
        out_shape=jax.ShapeDtypeStruct((B, L, D), jnp.float32),
        compiler_params=pltpu.CompilerParams(
            dimension_semantics=("parallel", "arbitrary"),
        ),
    )(seq_lens, sequences)
    return out, seq_lens
